# baseline (speedup 1.0000x reference)
import functools

import jax
import jax.numpy as jnp
from jax import lax
from jax.experimental import pallas as pl
from jax.experimental.pallas import tpu as pltpu, tpu_sc as plsc

N = 10000
E = 320000
NC, NS, L = 2, 16, 16
NW = NC * NS
CH = 128
STEPS = 80
EPAD = NW * STEPS * CH
NR = 10240
ROWS = NR // NS
_SET = 5
_G = STEPS // _SET

_MESH = plsc.VectorSubcoreMesh(
    core_axis_name="c", subcore_axis_name="s", num_cores=NC, num_subcores=NS)
_SC_PARAMS = pltpu.CompilerParams(use_tc_tiling_on_sc=False)


def _zero_rows(buf, nrows, ncols):
  zv = jnp.zeros((16,), jnp.float32)
  def zrow(r, carry):
    for c in range(ncols // 16):
      buf[r, pl.ds(c * 16, 16)] = zv
    return carry
  lax.fori_loop(0, nrows, zrow, 0)


@functools.partial(
    pl.kernel,
    out_type=jax.ShapeDtypeStruct((NC, NR), jnp.float32),
    mesh=_MESH,
    compiler_params=_SC_PARAMS,
    scratch_types=[
        pltpu.VMEM((STEPS, CH), jnp.int32),
        pltpu.VMEM((CH,), jnp.float32),
        pltpu.VMEM((ROWS,), jnp.float32),
        pltpu.VMEM_SHARED((NR,), jnp.float32),
    ],
)
def _sc_degree(dst_hbm, out_hbm, dst_v, ones_v, stage_v, hist_sh):
  cid = lax.axis_index("c")
  sid = lax.axis_index("s")
  wid = sid * NC + cid
  pltpu.sync_copy(dst_hbm.at[wid], dst_v)
  zv = jnp.zeros((16,), jnp.float32)
  ov = jnp.ones((16,), jnp.float32)
  def fill(r, carry):
    stage_v[pl.ds(r * 16, 16)] = zv
    return carry
  lax.fori_loop(0, ROWS // 16, fill, 0)
  for c in range(CH // 16):
    ones_v[pl.ds(c * 16, 16)] = ov
  pltpu.sync_copy(stage_v, hist_sh.at[pl.ds(sid * ROWS, ROWS)])
  plsc.subcore_barrier()

  def step(j, carry):
    pltpu.sync_copy(ones_v, hist_sh.at[dst_v.at[j]], add=True)
    return carry
  lax.fori_loop(0, STEPS, step, 0)
  plsc.subcore_barrier()
  pltpu.sync_copy(hist_sh.at[pl.ds(sid * ROWS, ROWS)], stage_v)
  pltpu.sync_copy(stage_v, out_hbm.at[cid, pl.ds(sid * ROWS, ROWS)])


ESTEPS = EPAD // (NS * CH)
_G = ESTEPS // _SET


def _run_ring(num_groups, gstart, gwait, sstart, swait):
  for b in range(_SET):
    gstart(0, 0, b)
  for b in range(_SET):
    gwait(0, 0, b)
  for b in range(_SET):
    gstart(1, 1, b)
  for b in range(_SET):
    sstart(0, 0, b)

  def pair(gp, carry):
    g1 = 2 * gp + 1
    for b in range(_SET):
      gwait(g1, 1, b)
    for b in range(_SET):
      swait(g1 - 1, 0, b)
      gstart(g1 + 1, 0, b)
    for b in range(_SET):
      sstart(g1, 1, b)
    g0 = 2 * gp + 2
    for b in range(_SET):
      gwait(g0, 0, b)
    for b in range(_SET):
      swait(g0 - 1, 1, b)
      gstart(g0 + 1, 1, b)
    for b in range(_SET):
      sstart(g0, 0, b)
    return carry
  lax.fori_loop(0, (num_groups - 2) // 2, pair, 0)

  gl = num_groups - 1
  for b in range(_SET):
    gwait(gl, 1, b)
  for b in range(_SET):
    swait(gl - 1, 0, b)
  for b in range(_SET):
    sstart(gl, 1, b)
  for b in range(_SET):
    swait(gl, 1, b)


def _make_sc_gs(F):
  FH = F // 2
  @functools.partial(
      pl.kernel,
      out_type=jax.ShapeDtypeStruct((NC, NR, FH), jnp.float32),
      mesh=_MESH,
      compiler_params=_SC_PARAMS,
      scratch_types=(
          [pltpu.VMEM((ESTEPS, CH), jnp.int32)] * 2
          + [pltpu.VMEM((CH, FH), jnp.float32)] * (2 * _SET)
          + [pltpu.VMEM_SHARED((NR, FH), jnp.float32)] * 2
          + [pltpu.SemaphoreType.DMA] * (4 * _SET)
      ),
  )
  def gs(src_hbm, dst_hbm, ya_hbm, yb_hbm, out_hbm, src_v, dst_v, *rest):
    nb = 2 * _SET
    bufs = rest[0:nb]
    agg_sh = rest[nb]
    y_sh = rest[nb + 1]
    gsem = rest[nb + 2:nb + 2 + nb]
    ssem = rest[nb + 2 + nb:nb + 2 + 2 * nb]
    cid = lax.axis_index("c")
    sid = lax.axis_index("s")
    pltpu.sync_copy(src_hbm.at[sid], src_v)
    pltpu.sync_copy(dst_hbm.at[sid], dst_v)
    _zero_rows(bufs[0], CH, FH)
    for kk in range(ROWS // CH):
      r0 = sid * ROWS + kk * CH
      @pl.when(cid == 0)
      def _():
        pltpu.sync_copy(ya_hbm.at[pl.ds(r0, CH)], bufs[1])
      @pl.when(cid == 1)
      def _():
        pltpu.sync_copy(yb_hbm.at[pl.ds(r0, CH)], bufs[1])
      pltpu.sync_copy(bufs[1], y_sh.at[pl.ds(r0, CH)])
      pltpu.sync_copy(bufs[0], agg_sh.at[pl.ds(r0, CH)])
    plsc.subcore_barrier()

    def buf(s_idx, b):
      return bufs[s_idx * _SET + b]

    def gstart(g, s_idx, b):
      pltpu.async_copy(y_sh.at[src_v.at[g * _SET + b]], buf(s_idx, b),
                       gsem[s_idx * _SET + b])

    def gwait(g, s_idx, b):
      pltpu.make_async_copy(y_sh.at[src_v.at[g * _SET + b]], buf(s_idx, b),
                            gsem[s_idx * _SET + b]).wait()

    def sstart(g, s_idx, b):
      pltpu.async_copy(buf(s_idx, b), agg_sh.at[dst_v.at[g * _SET + b]],
                       ssem[s_idx * _SET + b], add=True)

    def swait(g, s_idx, b):
      pltpu.make_async_copy(buf(s_idx, b), agg_sh.at[dst_v.at[g * _SET + b]],
                            ssem[s_idx * _SET + b]).wait()

    _run_ring(_G, gstart, gwait, sstart, swait)

    plsc.subcore_barrier()
    for kk in range(ROWS // CH):
      r0 = sid * ROWS + kk * CH
      pltpu.sync_copy(agg_sh.at[pl.ds(r0, CH)], bufs[0])
      pltpu.sync_copy(bufs[0], out_hbm.at[cid, pl.ds(r0, CH)])
  return gs


@functools.partial(
    pl.kernel,
    out_type=jax.ShapeDtypeStruct((NC, NR, 16), jnp.float32),
    mesh=_MESH,
    compiler_params=_SC_PARAMS,
    scratch_types=(
        [pltpu.VMEM((STEPS, CH), jnp.int32)] * 2
        + [pltpu.VMEM((CH, 16), jnp.float32)] * (2 * _SET)
        + [pltpu.VMEM_SHARED((NR, 16), jnp.float32)] * 2
        + [pltpu.SemaphoreType.DMA] * (4 * _SET)
    ),
)
def _sc_gs16(src_hbm, dst_hbm, y_hbm, out_hbm, src_v, dst_v, *rest):
  nb = 2 * _SET
  bufs = rest[0:nb]
  agg_sh = rest[nb]
  y_sh = rest[nb + 1]
  gsem = rest[nb + 2:nb + 2 + nb]
  ssem = rest[nb + 2 + nb:nb + 2 + 2 * nb]
  cid = lax.axis_index("c")
  sid = lax.axis_index("s")
  wid = sid * NC + cid
  pltpu.sync_copy(src_hbm.at[wid], src_v)
  pltpu.sync_copy(dst_hbm.at[wid], dst_v)
  _zero_rows(bufs[0], CH, 16)
  for kk in range(ROWS // CH):
    r0 = sid * ROWS + kk * CH
    pltpu.sync_copy(y_hbm.at[pl.ds(r0, CH)], bufs[1])
    pltpu.sync_copy(bufs[1], y_sh.at[pl.ds(r0, CH)])
    pltpu.sync_copy(bufs[0], agg_sh.at[pl.ds(r0, CH)])
  plsc.subcore_barrier()

  def buf(s_idx, b):
    return bufs[s_idx * _SET + b]

  def gstart(g, s_idx, b):
    pltpu.async_copy(y_sh.at[src_v.at[g * _SET + b]], buf(s_idx, b),
                     gsem[s_idx * _SET + b])

  def gwait(g, s_idx, b):
    pltpu.make_async_copy(y_sh.at[src_v.at[g * _SET + b]], buf(s_idx, b),
                          gsem[s_idx * _SET + b]).wait()

  def sstart(g, s_idx, b):
    pltpu.async_copy(buf(s_idx, b), agg_sh.at[dst_v.at[g * _SET + b]],
                     ssem[s_idx * _SET + b], add=True)

  def swait(g, s_idx, b):
    pltpu.make_async_copy(buf(s_idx, b), agg_sh.at[dst_v.at[g * _SET + b]],
                          ssem[s_idx * _SET + b]).wait()

  _run_ring(STEPS // _SET, gstart, gwait, sstart, swait)

  plsc.subcore_barrier()
  for kk in range(ROWS // CH):
    r0 = sid * ROWS + kk * CH
    pltpu.sync_copy(agg_sh.at[pl.ds(r0, CH)], bufs[0])
    pltpu.sync_copy(bufs[0], out_hbm.at[cid, pl.ds(r0, CH)])


_sc_gs64 = _make_sc_gs(64)
_sc_gs32 = _make_sc_gs(32)


@functools.partial(
    pl.kernel,
    out_type=[jax.ShapeDtypeStruct((EPAD, 16), jnp.float32),
              jax.ShapeDtypeStruct((EPAD, 16), jnp.float32)],
    mesh=_MESH,
    compiler_params=_SC_PARAMS,
    scratch_types=(
        [pltpu.VMEM((STEPS, CH), jnp.int32)] * 2
        + [pltpu.VMEM((ROWS, 16), jnp.float32)] * 3
        + [pltpu.VMEM((ROWS, 16), jnp.float32)]
        + [pltpu.VMEM((1, 16), jnp.float32)]
        + [pltpu.VMEM((ROWS, 16), jnp.float32)]
        + [pltpu.VMEM_SHARED((NR, 16), jnp.float32)]
        + [pltpu.VMEM((CH, 16), jnp.float32)] * 4
        + [pltpu.SemaphoreType.DMA] * 8
    ),
)
def _sc_edge(src_hbm, dst_hbm, aggp_hbm, y3_hbm, disb_hbm, c3_hbm,
             outs_hbm, outd_hbm, src_v, dst_v, a0_v, a1_v, y3_v, disb_v, c3_v,
             h3_v, h3_sh, *rest):
  bufs = rest[0:4]
  gsem = rest[4:8]
  wsem = rest[8:12]
  cid = lax.axis_index("c")
  sid = lax.axis_index("s")
  wid = sid * NC + cid
  pltpu.sync_copy(src_hbm.at[wid], src_v)
  pltpu.sync_copy(dst_hbm.at[wid], dst_v)
  r0 = sid * ROWS
  pltpu.sync_copy(aggp_hbm.at[0, pl.ds(r0, ROWS)], a0_v)
  pltpu.sync_copy(aggp_hbm.at[1, pl.ds(r0, ROWS)], a1_v)
  pltpu.sync_copy(y3_hbm.at[pl.ds(r0, ROWS)], y3_v)
  pltpu.sync_copy(disb_hbm.at[pl.ds(r0, ROWS)], disb_v)
  pltpu.sync_copy(c3_hbm, c3_v)
  c3 = c3_v[0, :]
  def row(r, carry):
    t = (a0_v[r, :] + a1_v[r, :] + y3_v[r, :]) * disb_v[r, :] + c3
    h3_v[r, :] = jnp.maximum(t, 0.0)
    return carry
  lax.fori_loop(0, ROWS, row, 0)
  pltpu.sync_copy(h3_v, h3_sh.at[pl.ds(r0, ROWS)])
  plsc.subcore_barrier()

  base = wid * STEPS

  def gstart(j, p, b):
    idx = src_v if p == 0 else dst_v
    pltpu.async_copy(h3_sh.at[idx.at[j]], bufs[2 * p + b], gsem[2 * p + b])

  def gwait(j, p, b):
    idx = src_v if p == 0 else dst_v
    pltpu.make_async_copy(h3_sh.at[idx.at[j]], bufs[2 * p + b],
                          gsem[2 * p + b]).wait()

  def wstart(j, p, b):
    out = outs_hbm if p == 0 else outd_hbm
    pltpu.async_copy(bufs[2 * p + b], out.at[pl.ds((base + j) * CH, CH)],
                     wsem[2 * p + b])

  def wwait(j, p, b):
    out = outs_hbm if p == 0 else outd_hbm
    pltpu.make_async_copy(bufs[2 * p + b], out.at[pl.ds((base + j) * CH, CH)],
                          wsem[2 * p + b]).wait()

  for p in range(2):
    gstart(0, p, 0)
    gstart(1, p, 1)

  def step(i, carry):
    for b in range(2):
      j = 2 * i + b
      for p in range(2):
        gwait(j, p, b)
        wstart(j, p, b)
      for p in range(2):
        wwait(j, p, b)
        gstart(j + 2, p, b)
    return carry
  lax.fori_loop(0, STEPS // 2 - 1, step, 0)
  for b in range(2):
    j = STEPS - 2 + b
    for p in range(2):
      gwait(j, p, b)
      wstart(j, p, b)
    for p in range(2):
      wwait(j, p, b)


def _dot(a, b):
  return lax.dot_general(a, b, (((1,), (0,)), ((), ())),
                         preferred_element_type=jnp.float32)


def _tc_prep_body(degp_ref, x_ref, w_ref, b3_ref, g3_ref, be3_ref, rm3_ref,
                  rv3_ref, ya_ref, yb_ref, dis_ref, disb_ref, c3_ref):
  deg = degp_ref[0, :] + degp_ref[1, :] + 1.0
  dis = lax.rsqrt(deg)
  dis_ref[...] = dis
  y = _dot(x_ref[...], w_ref[...]) * dis[:, None]
  fh = y.shape[1] // 2
  ya_ref[...] = y[:, :fh]
  yb_ref[...] = y[:, fh:]
  a3 = g3_ref[...] * lax.rsqrt(rv3_ref[...] + 1e-5)
  disb_ref[...] = dis[:, None] * a3
  c3_ref[...] = (b3_ref[...] - rm3_ref[...]) * a3 + be3_ref[...]


def _tc_prep(degp, x, w, b3, g3, be3, rm3, rv3):
  fh = w.shape[1] // 2
  return pl.pallas_call(
      _tc_prep_body,
      out_shape=[jax.ShapeDtypeStruct((NR, fh), jnp.float32),
                 jax.ShapeDtypeStruct((NR, fh), jnp.float32),
                 jax.ShapeDtypeStruct((NR,), jnp.float32),
                 jax.ShapeDtypeStruct((NR, 16), jnp.float32),
                 jax.ShapeDtypeStruct((1, 16), jnp.float32)],
  )(degp, x, w, b3[None, :], g3[None, :], be3[None, :], rm3[None, :],
    rv3[None, :])


def _bn_relu(t, g, be, rm, rv):
  scale = g * lax.rsqrt(rv + 1e-5)
  return jnp.maximum((t - rm) * scale + be, 0.0)


def _make_tc_layer_body(split_out):
  def body(aggp_ref, ya_ref, yb_ref, dis_ref, b_ref, g_ref, be_ref, rm_ref,
           rv_ref, w_ref, *outs):
    dis = dis_ref[...][:, None]
    t = jnp.concatenate(
        [aggp_ref[0] + ya_ref[...], aggp_ref[1] + yb_ref[...]], axis=1)
    t = t * dis + b_ref[...]
    h = _bn_relu(t, g_ref[...], be_ref[...], rm_ref[...], rv_ref[...])
    yn = _dot(h, w_ref[...]) * dis
    if split_out:
      fh = yn.shape[1] // 2
      outs[0][...] = yn[:, :fh]
      outs[1][...] = yn[:, fh:]
    else:
      outs[0][...] = yn
  return body


def _tc_layer(aggp, ya, yb, dis, b, g, be, rm, rv, w, split_out):
  fo = w.shape[1]
  if split_out:
    osh = [jax.ShapeDtypeStruct((NR, fo // 2), jnp.float32)] * 2
  else:
    osh = jax.ShapeDtypeStruct((NR, fo), jnp.float32)
  return pl.pallas_call(
      _make_tc_layer_body(split_out),
      out_shape=osh,
  )(aggp, ya, yb, dis, b[None, :], g[None, :], be[None, :], rm[None, :],
    rv[None, :], w)


_EBLK = 6400


def _tc_mlp_body(hs_ref, hd_ref, ea_ref, bda_ref, bdb_ref, bdc_ref, b1_ref,
                 bd2_ref, b2_ref, bd3_ref, b3_ref, out_ref):
  z = (_dot(hs_ref[...], bda_ref[...]) + _dot(hd_ref[...], bdb_ref[...]) +
       _dot(ea_ref[...], bdc_ref[...]) + b1_ref[...])
  z = jnp.maximum(z, 0.0)
  z = jnp.maximum(_dot(z, bd2_ref[...]) + b2_ref[...], 0.0)
  o_p = _dot(z, bd3_ref[...]) + b3_ref[...]
  o_big = jnp.broadcast_to(o_p[:, None, :], (_EBLK // 8, 8, 16))
  o_big = o_big.reshape(_EBLK, 16)
  k = jax.lax.broadcasted_iota(jnp.int32, (8, 16), 0)
  l = jax.lax.broadcasted_iota(jnp.int32, (8, 16), 1)
  m0 = jnp.where(l == 2 * k, 1.0, 0.0)
  m1 = jnp.where(l == 2 * k + 1, 1.0, 0.0)
  m0t = jnp.tile(m0, (_EBLK // 8, 1))
  m1t = jnp.tile(m1, (_EBLK // 8, 1))
  c0 = jnp.sum(o_big * m0t, axis=1)
  c1 = jnp.sum(o_big * m1t, axis=1)
  out_ref[...] = jnp.concatenate([c0[:, None], c1[:, None]], axis=1)


def _mlp_prep(ea, w1, b1, w2, b2, w3, b3):
  eye8 = jnp.eye(8, dtype=jnp.float32)
  bda = jnp.kron(eye8, w1[0:16])
  bdb = jnp.kron(eye8, w1[16:32])
  bdc = jnp.kron(eye8, w1[32:48])
  bd2 = jnp.kron(eye8, w2)
  bd3 = jnp.kron(eye8, w3)
  b1p = jnp.tile(b1, 8)[None, :]
  b2p = jnp.tile(b2, 8)[None, :]
  b3p = jnp.tile(b3, 8)[None, :]
  ea_p = ea.reshape(E // 8, 128)
  return ea_p, (bda, bdb, bdc, b1p, bd2, b2p, bd3, b3p)


def _tc_mlp(hs, hd, ea_p, mlpw):
  bda, bdb, bdc, b1p, bd2, b2p, bd3, b3p = mlpw
  hs_p = hs.reshape(EPAD // 8, 128)
  hd_p = hd.reshape(EPAD // 8, 128)
  grid = E // _EBLK
  blk = lambda i: (i, 0)
  full = lambda i: (0, 0)
  out_p = pl.pallas_call(
      _tc_mlp_body,
      grid=(grid,),
      in_specs=[
          pl.BlockSpec((_EBLK // 8, 128), blk),
          pl.BlockSpec((_EBLK // 8, 128), blk),
          pl.BlockSpec((_EBLK // 8, 128), blk),
          pl.BlockSpec((128, 256), full),
          pl.BlockSpec((128, 256), full),
          pl.BlockSpec((128, 256), full),
          pl.BlockSpec((1, 256), full),
          pl.BlockSpec((256, 128), full),
          pl.BlockSpec((1, 128), full),
          pl.BlockSpec((128, 16), full),
          pl.BlockSpec((1, 16), full),
      ],
      out_specs=pl.BlockSpec((_EBLK, 2), blk),
      out_shape=jax.ShapeDtypeStruct((E, 2), jnp.float32),
  )(hs_p, hd_p, ea_p, bda, bdb, bdc, b1p, bd2, b2p, bd3, b3p)
  return out_p


def kernel(x, edge_index, edge_attr, W1, b1, g1, be1, rm1, rv1,
           W2, b2, g2, be2, rm2, rv2, W3, b3, g3, be3, rm3, rv3,
           ecW1, ecb1, ecW2, ecb2, ecW3, ecb3):
  src = edge_index[0]
  dst = edge_index[1]
  src_pad = jnp.concatenate([src, jnp.zeros((EPAD - E,), jnp.int32)])
  dst_pad = jnp.concatenate([dst, jnp.full((EPAD - E,), N, jnp.int32)])
  src_p = src_pad.reshape(NW, STEPS, CH)
  dst_p = dst_pad.reshape(NW, STEPS, CH)
  src_s = src_pad.reshape(NS, ESTEPS, CH)
  dst_s = dst_pad.reshape(NS, ESTEPS, CH)
  x_p = jnp.pad(x, ((0, NR - N), (0, 0)))
  ea_p, mlpw = _mlp_prep(edge_attr, ecW1, ecb1, ecW2, ecb2, ecW3, ecb3)

  degp = _sc_degree(dst_p)
  y1a, y1b, dis, disb, c3 = _tc_prep(degp, x_p, W1, b3, g3, be3, rm3, rv3)
  agg1 = _sc_gs64(src_s, dst_s, y1a, y1b)
  y2a, y2b = _tc_layer(agg1, y1a, y1b, dis, b1, g1, be1, rm1, rv1, W2, True)
  agg2 = _sc_gs32(src_s, dst_s, y2a, y2b)
  y3 = _tc_layer(agg2, y2a, y2b, dis, b2, g2, be2, rm2, rv2, W3, False)
  agg3p = _sc_gs16(src_p, dst_p, y3)
  hs, hd = _sc_edge(src_p, dst_p, agg3p, y3, disb, c3)
  return _tc_mlp(hs, hd, ea_p, mlpw)


# speedup vs baseline: 1.0167x; 1.0167x over previous
import functools

import jax
import jax.numpy as jnp
from jax import lax
from jax.experimental import pallas as pl
from jax.experimental.pallas import tpu as pltpu, tpu_sc as plsc

N = 10000
E = 320000
NC, NS, L = 2, 16, 16
NW = NC * NS
CH = 128
STEPS = 80
EPAD = NW * STEPS * CH
NR = 10240
ROWS = NR // NS
_SET = 5
_G = STEPS // _SET

_MESH = plsc.VectorSubcoreMesh(
    core_axis_name="c", subcore_axis_name="s", num_cores=NC, num_subcores=NS)
_SC_PARAMS = pltpu.CompilerParams(use_tc_tiling_on_sc=False)


def _zero_rows(buf, nrows, ncols):
  zv = jnp.zeros((16,), jnp.float32)
  def zrow(r, carry):
    for c in range(ncols // 16):
      buf[r, pl.ds(c * 16, 16)] = zv
    return carry
  lax.fori_loop(0, nrows, zrow, 0)


@functools.partial(
    pl.kernel,
    out_type=jax.ShapeDtypeStruct((NC, NR), jnp.float32),
    mesh=_MESH,
    compiler_params=_SC_PARAMS,
    scratch_types=[
        pltpu.VMEM((STEPS, CH), jnp.int32),
        pltpu.VMEM((CH,), jnp.float32),
        pltpu.VMEM((ROWS,), jnp.float32),
        pltpu.VMEM_SHARED((NR,), jnp.float32),
    ],
)
def _sc_degree(dst_hbm, out_hbm, dst_v, ones_v, stage_v, hist_sh):
  cid = lax.axis_index("c")
  sid = lax.axis_index("s")
  wid = sid * NC + cid
  pltpu.sync_copy(dst_hbm.at[wid], dst_v)
  zv = jnp.zeros((16,), jnp.float32)
  ov = jnp.ones((16,), jnp.float32)
  def fill(r, carry):
    stage_v[pl.ds(r * 16, 16)] = zv
    return carry
  lax.fori_loop(0, ROWS // 16, fill, 0)
  for c in range(CH // 16):
    ones_v[pl.ds(c * 16, 16)] = ov
  pltpu.sync_copy(stage_v, hist_sh.at[pl.ds(sid * ROWS, ROWS)])
  plsc.subcore_barrier()

  def step(j, carry):
    pltpu.sync_copy(ones_v, hist_sh.at[dst_v.at[j]], add=True)
    return carry
  lax.fori_loop(0, STEPS, step, 0)
  plsc.subcore_barrier()
  pltpu.sync_copy(hist_sh.at[pl.ds(sid * ROWS, ROWS)], stage_v)
  pltpu.sync_copy(stage_v, out_hbm.at[cid, pl.ds(sid * ROWS, ROWS)])


ESTEPS = EPAD // (NS * CH)
_G = ESTEPS // _SET


def _run_ring(num_groups, gstart, gwait, sstart, swait):
  for b in range(_SET):
    gstart(0, 0, b)
  for b in range(_SET):
    gwait(0, 0, b)
  for b in range(_SET):
    gstart(1, 1, b)
  for b in range(_SET):
    sstart(0, 0, b)

  def pair(gp, carry):
    g1 = 2 * gp + 1
    for b in range(_SET):
      gwait(g1, 1, b)
    for b in range(_SET):
      swait(g1 - 1, 0, b)
      gstart(g1 + 1, 0, b)
    for b in range(_SET):
      sstart(g1, 1, b)
    g0 = 2 * gp + 2
    for b in range(_SET):
      gwait(g0, 0, b)
    for b in range(_SET):
      swait(g0 - 1, 1, b)
      gstart(g0 + 1, 1, b)
    for b in range(_SET):
      sstart(g0, 0, b)
    return carry
  lax.fori_loop(0, (num_groups - 2) // 2, pair, 0)

  gl = num_groups - 1
  for b in range(_SET):
    gwait(gl, 1, b)
  for b in range(_SET):
    swait(gl - 1, 0, b)
  for b in range(_SET):
    sstart(gl, 1, b)
  for b in range(_SET):
    swait(gl, 1, b)


def _make_sc_gs(F):
  FH = F // 2
  @functools.partial(
      pl.kernel,
      out_type=jax.ShapeDtypeStruct((NC, NR, FH), jnp.float32),
      mesh=_MESH,
      compiler_params=_SC_PARAMS,
      scratch_types=(
          [pltpu.VMEM((ESTEPS, CH), jnp.int32)] * 2
          + [pltpu.VMEM((CH, FH), jnp.float32)] * (2 * _SET)
          + [pltpu.VMEM_SHARED((NR, FH), jnp.float32)] * 2
          + [pltpu.SemaphoreType.DMA] * (4 * _SET)
      ),
  )
  def gs(src_hbm, dst_hbm, ya_hbm, yb_hbm, out_hbm, src_v, dst_v, *rest):
    nb = 2 * _SET
    bufs = rest[0:nb]
    agg_sh = rest[nb]
    y_sh = rest[nb + 1]
    gsem = rest[nb + 2:nb + 2 + nb]
    ssem = rest[nb + 2 + nb:nb + 2 + 2 * nb]
    cid = lax.axis_index("c")
    sid = lax.axis_index("s")
    pltpu.sync_copy(src_hbm.at[sid], src_v)
    pltpu.sync_copy(dst_hbm.at[sid], dst_v)
    _zero_rows(bufs[0], CH, FH)
    NK = ROWS // CH
    for kk in range(NK):
      r0 = sid * ROWS + kk * CH
      pltpu.async_copy(bufs[0], agg_sh.at[pl.ds(r0, CH)], ssem[kk])
      @pl.when(cid == 0)
      def _():
        pltpu.async_copy(ya_hbm.at[pl.ds(r0, CH)], bufs[1 + kk], gsem[kk])
      @pl.when(cid == 1)
      def _():
        pltpu.async_copy(yb_hbm.at[pl.ds(r0, CH)], bufs[1 + kk], gsem[kk])
    for kk in range(NK):
      r0 = sid * ROWS + kk * CH
      @pl.when(cid == 0)
      def _():
        pltpu.make_async_copy(ya_hbm.at[pl.ds(r0, CH)], bufs[1 + kk],
                              gsem[kk]).wait()
      @pl.when(cid == 1)
      def _():
        pltpu.make_async_copy(yb_hbm.at[pl.ds(r0, CH)], bufs[1 + kk],
                              gsem[kk]).wait()
      pltpu.async_copy(bufs[1 + kk], y_sh.at[pl.ds(r0, CH)], ssem[NK + kk])
    for kk in range(NK):
      r0 = sid * ROWS + kk * CH
      pltpu.make_async_copy(bufs[0], agg_sh.at[pl.ds(r0, CH)],
                            ssem[kk]).wait()
      pltpu.make_async_copy(bufs[1 + kk], y_sh.at[pl.ds(r0, CH)],
                            ssem[NK + kk]).wait()
    plsc.subcore_barrier()

    def buf(s_idx, b):
      return bufs[s_idx * _SET + b]

    def gstart(g, s_idx, b):
      pltpu.async_copy(y_sh.at[src_v.at[g * _SET + b]], buf(s_idx, b),
                       gsem[s_idx * _SET + b])

    def gwait(g, s_idx, b):
      pltpu.make_async_copy(y_sh.at[src_v.at[g * _SET + b]], buf(s_idx, b),
                            gsem[s_idx * _SET + b]).wait()

    def sstart(g, s_idx, b):
      pltpu.async_copy(buf(s_idx, b), agg_sh.at[dst_v.at[g * _SET + b]],
                       ssem[s_idx * _SET + b], add=True)

    def swait(g, s_idx, b):
      pltpu.make_async_copy(buf(s_idx, b), agg_sh.at[dst_v.at[g * _SET + b]],
                            ssem[s_idx * _SET + b]).wait()

    _run_ring(_G, gstart, gwait, sstart, swait)

    plsc.subcore_barrier()
    for kk in range(ROWS // CH):
      r0 = sid * ROWS + kk * CH
      pltpu.sync_copy(agg_sh.at[pl.ds(r0, CH)], bufs[0])
      pltpu.sync_copy(bufs[0], out_hbm.at[cid, pl.ds(r0, CH)])
  return gs


@functools.partial(
    pl.kernel,
    out_type=jax.ShapeDtypeStruct((NC, NR, 16), jnp.float32),
    mesh=_MESH,
    compiler_params=_SC_PARAMS,
    scratch_types=(
        [pltpu.VMEM((STEPS, CH), jnp.int32)] * 2
        + [pltpu.VMEM((CH, 16), jnp.float32)] * (2 * _SET)
        + [pltpu.VMEM_SHARED((NR, 16), jnp.float32)] * 2
        + [pltpu.SemaphoreType.DMA] * (4 * _SET)
    ),
)
def _sc_gs16(src_hbm, dst_hbm, y_hbm, out_hbm, src_v, dst_v, *rest):
  nb = 2 * _SET
  bufs = rest[0:nb]
  agg_sh = rest[nb]
  y_sh = rest[nb + 1]
  gsem = rest[nb + 2:nb + 2 + nb]
  ssem = rest[nb + 2 + nb:nb + 2 + 2 * nb]
  cid = lax.axis_index("c")
  sid = lax.axis_index("s")
  wid = sid * NC + cid
  pltpu.sync_copy(src_hbm.at[wid], src_v)
  pltpu.sync_copy(dst_hbm.at[wid], dst_v)
  _zero_rows(bufs[0], CH, 16)
  NK = ROWS // CH
  for kk in range(NK):
    r0 = sid * ROWS + kk * CH
    pltpu.async_copy(bufs[0], agg_sh.at[pl.ds(r0, CH)], ssem[kk])
    pltpu.async_copy(y_hbm.at[pl.ds(r0, CH)], bufs[1 + kk], gsem[kk])
  for kk in range(NK):
    r0 = sid * ROWS + kk * CH
    pltpu.make_async_copy(y_hbm.at[pl.ds(r0, CH)], bufs[1 + kk],
                          gsem[kk]).wait()
    pltpu.async_copy(bufs[1 + kk], y_sh.at[pl.ds(r0, CH)], ssem[NK + kk])
  for kk in range(NK):
    r0 = sid * ROWS + kk * CH
    pltpu.make_async_copy(bufs[0], agg_sh.at[pl.ds(r0, CH)], ssem[kk]).wait()
    pltpu.make_async_copy(bufs[1 + kk], y_sh.at[pl.ds(r0, CH)],
                          ssem[NK + kk]).wait()
  plsc.subcore_barrier()

  def buf(s_idx, b):
    return bufs[s_idx * _SET + b]

  def gstart(g, s_idx, b):
    pltpu.async_copy(y_sh.at[src_v.at[g * _SET + b]], buf(s_idx, b),
                     gsem[s_idx * _SET + b])

  def gwait(g, s_idx, b):
    pltpu.make_async_copy(y_sh.at[src_v.at[g * _SET + b]], buf(s_idx, b),
                          gsem[s_idx * _SET + b]).wait()

  def sstart(g, s_idx, b):
    pltpu.async_copy(buf(s_idx, b), agg_sh.at[dst_v.at[g * _SET + b]],
                     ssem[s_idx * _SET + b], add=True)

  def swait(g, s_idx, b):
    pltpu.make_async_copy(buf(s_idx, b), agg_sh.at[dst_v.at[g * _SET + b]],
                          ssem[s_idx * _SET + b]).wait()

  _run_ring(STEPS // _SET, gstart, gwait, sstart, swait)

  plsc.subcore_barrier()
  for kk in range(ROWS // CH):
    r0 = sid * ROWS + kk * CH
    pltpu.sync_copy(agg_sh.at[pl.ds(r0, CH)], bufs[0])
    pltpu.sync_copy(bufs[0], out_hbm.at[cid, pl.ds(r0, CH)])


_sc_gs64 = _make_sc_gs(64)
_sc_gs32 = _make_sc_gs(32)


@functools.partial(
    pl.kernel,
    out_type=[jax.ShapeDtypeStruct((EPAD, 16), jnp.float32),
              jax.ShapeDtypeStruct((EPAD, 16), jnp.float32)],
    mesh=_MESH,
    compiler_params=_SC_PARAMS,
    scratch_types=(
        [pltpu.VMEM((STEPS, CH), jnp.int32)] * 2
        + [pltpu.VMEM((ROWS, 16), jnp.float32)] * 3
        + [pltpu.VMEM((ROWS, 16), jnp.float32)]
        + [pltpu.VMEM((1, 16), jnp.float32)]
        + [pltpu.VMEM((ROWS, 16), jnp.float32)]
        + [pltpu.VMEM_SHARED((NR, 16), jnp.float32)]
        + [pltpu.VMEM((CH, 16), jnp.float32)] * 4
        + [pltpu.SemaphoreType.DMA] * 8
    ),
)
def _sc_edge(src_hbm, dst_hbm, aggp_hbm, y3_hbm, disb_hbm, c3_hbm,
             outs_hbm, outd_hbm, src_v, dst_v, a0_v, a1_v, y3_v, disb_v, c3_v,
             h3_v, h3_sh, *rest):
  bufs = rest[0:4]
  gsem = rest[4:8]
  wsem = rest[8:12]
  cid = lax.axis_index("c")
  sid = lax.axis_index("s")
  wid = sid * NC + cid
  pltpu.sync_copy(src_hbm.at[wid], src_v)
  pltpu.sync_copy(dst_hbm.at[wid], dst_v)
  r0 = sid * ROWS
  pltpu.sync_copy(aggp_hbm.at[0, pl.ds(r0, ROWS)], a0_v)
  pltpu.sync_copy(aggp_hbm.at[1, pl.ds(r0, ROWS)], a1_v)
  pltpu.sync_copy(y3_hbm.at[pl.ds(r0, ROWS)], y3_v)
  pltpu.sync_copy(disb_hbm.at[pl.ds(r0, ROWS)], disb_v)
  pltpu.sync_copy(c3_hbm, c3_v)
  c3 = c3_v[0, :]
  def row(r, carry):
    t = (a0_v[r, :] + a1_v[r, :] + y3_v[r, :]) * disb_v[r, :] + c3
    h3_v[r, :] = jnp.maximum(t, 0.0)
    return carry
  lax.fori_loop(0, ROWS, row, 0)
  pltpu.sync_copy(h3_v, h3_sh.at[pl.ds(r0, ROWS)])
  plsc.subcore_barrier()

  base = wid * STEPS

  def gstart(j, p, b):
    idx = src_v if p == 0 else dst_v
    pltpu.async_copy(h3_sh.at[idx.at[j]], bufs[2 * p + b], gsem[2 * p + b])

  def gwait(j, p, b):
    idx = src_v if p == 0 else dst_v
    pltpu.make_async_copy(h3_sh.at[idx.at[j]], bufs[2 * p + b],
                          gsem[2 * p + b]).wait()

  def wstart(j, p, b):
    out = outs_hbm if p == 0 else outd_hbm
    pltpu.async_copy(bufs[2 * p + b], out.at[pl.ds((base + j) * CH, CH)],
                     wsem[2 * p + b])

  def wwait(j, p, b):
    out = outs_hbm if p == 0 else outd_hbm
    pltpu.make_async_copy(bufs[2 * p + b], out.at[pl.ds((base + j) * CH, CH)],
                          wsem[2 * p + b]).wait()

  for p in range(2):
    gstart(0, p, 0)
    gstart(1, p, 1)

  def step(i, carry):
    for b in range(2):
      j = 2 * i + b
      for p in range(2):
        gwait(j, p, b)
        wstart(j, p, b)
      for p in range(2):
        wwait(j, p, b)
        gstart(j + 2, p, b)
    return carry
  lax.fori_loop(0, STEPS // 2 - 1, step, 0)
  for b in range(2):
    j = STEPS - 2 + b
    for p in range(2):
      gwait(j, p, b)
      wstart(j, p, b)
    for p in range(2):
      wwait(j, p, b)


def _dot(a, b):
  return lax.dot_general(a, b, (((1,), (0,)), ((), ())),
                         preferred_element_type=jnp.float32)


def _tc_prep_body(degp_ref, x_ref, w_ref, b3_ref, g3_ref, be3_ref, rm3_ref,
                  rv3_ref, ya_ref, yb_ref, dis_ref, disb_ref, c3_ref):
  deg = degp_ref[0, :] + degp_ref[1, :] + 1.0
  dis = lax.rsqrt(deg)
  dis_ref[...] = dis
  y = _dot(x_ref[...], w_ref[...]) * dis[:, None]
  fh = y.shape[1] // 2
  ya_ref[...] = y[:, :fh]
  yb_ref[...] = y[:, fh:]
  a3 = g3_ref[...] * lax.rsqrt(rv3_ref[...] + 1e-5)
  disb_ref[...] = dis[:, None] * a3
  c3_ref[...] = (b3_ref[...] - rm3_ref[...]) * a3 + be3_ref[...]


def _tc_prep(degp, x, w, b3, g3, be3, rm3, rv3):
  fh = w.shape[1] // 2
  return pl.pallas_call(
      _tc_prep_body,
      out_shape=[jax.ShapeDtypeStruct((NR, fh), jnp.float32),
                 jax.ShapeDtypeStruct((NR, fh), jnp.float32),
                 jax.ShapeDtypeStruct((NR,), jnp.float32),
                 jax.ShapeDtypeStruct((NR, 16), jnp.float32),
                 jax.ShapeDtypeStruct((1, 16), jnp.float32)],
  )(degp, x, w, b3[None, :], g3[None, :], be3[None, :], rm3[None, :],
    rv3[None, :])


def _bn_relu(t, g, be, rm, rv):
  scale = g * lax.rsqrt(rv + 1e-5)
  return jnp.maximum((t - rm) * scale + be, 0.0)


def _make_tc_layer_body(split_out):
  def body(aggp_ref, ya_ref, yb_ref, dis_ref, b_ref, g_ref, be_ref, rm_ref,
           rv_ref, w_ref, *outs):
    dis = dis_ref[...][:, None]
    t = jnp.concatenate(
        [aggp_ref[0] + ya_ref[...], aggp_ref[1] + yb_ref[...]], axis=1)
    t = t * dis + b_ref[...]
    h = _bn_relu(t, g_ref[...], be_ref[...], rm_ref[...], rv_ref[...])
    yn = _dot(h, w_ref[...]) * dis
    if split_out:
      fh = yn.shape[1] // 2
      outs[0][...] = yn[:, :fh]
      outs[1][...] = yn[:, fh:]
    else:
      outs[0][...] = yn
  return body


def _tc_layer(aggp, ya, yb, dis, b, g, be, rm, rv, w, split_out):
  fo = w.shape[1]
  if split_out:
    osh = [jax.ShapeDtypeStruct((NR, fo // 2), jnp.float32)] * 2
  else:
    osh = jax.ShapeDtypeStruct((NR, fo), jnp.float32)
  return pl.pallas_call(
      _make_tc_layer_body(split_out),
      out_shape=osh,
  )(aggp, ya, yb, dis, b[None, :], g[None, :], be[None, :], rm[None, :],
    rv[None, :], w)


_EBLK = 6400


def _tc_mlp_body(hs_ref, hd_ref, ea_ref, bda_ref, bdb_ref, bdc_ref, b1_ref,
                 bd2_ref, b2_ref, bd3_ref, b3_ref, out_ref):
  z = (_dot(hs_ref[...], bda_ref[...]) + _dot(hd_ref[...], bdb_ref[...]) +
       _dot(ea_ref[...], bdc_ref[...]) + b1_ref[...])
  z = jnp.maximum(z, 0.0)
  z = jnp.maximum(_dot(z, bd2_ref[...]) + b2_ref[...], 0.0)
  o_p = _dot(z, bd3_ref[...]) + b3_ref[...]
  o_big = jnp.broadcast_to(o_p[:, None, :], (_EBLK // 8, 8, 16))
  o_big = o_big.reshape(_EBLK, 16)
  k = jax.lax.broadcasted_iota(jnp.int32, (8, 16), 0)
  l = jax.lax.broadcasted_iota(jnp.int32, (8, 16), 1)
  m0 = jnp.where(l == 2 * k, 1.0, 0.0)
  m1 = jnp.where(l == 2 * k + 1, 1.0, 0.0)
  m0t = jnp.tile(m0, (_EBLK // 8, 1))
  m1t = jnp.tile(m1, (_EBLK // 8, 1))
  c0 = jnp.sum(o_big * m0t, axis=1)
  c1 = jnp.sum(o_big * m1t, axis=1)
  out_ref[...] = jnp.concatenate([c0[:, None], c1[:, None]], axis=1)


def _mlp_prep(ea, w1, b1, w2, b2, w3, b3):
  eye8 = jnp.eye(8, dtype=jnp.float32)
  bda = jnp.kron(eye8, w1[0:16])
  bdb = jnp.kron(eye8, w1[16:32])
  bdc = jnp.kron(eye8, w1[32:48])
  bd2 = jnp.kron(eye8, w2)
  bd3 = jnp.kron(eye8, w3)
  b1p = jnp.tile(b1, 8)[None, :]
  b2p = jnp.tile(b2, 8)[None, :]
  b3p = jnp.tile(b3, 8)[None, :]
  ea_p = ea.reshape(E // 8, 128)
  return ea_p, (bda, bdb, bdc, b1p, bd2, b2p, bd3, b3p)


def _tc_mlp(hs, hd, ea_p, mlpw):
  bda, bdb, bdc, b1p, bd2, b2p, bd3, b3p = mlpw
  hs_p = hs.reshape(EPAD // 8, 128)
  hd_p = hd.reshape(EPAD // 8, 128)
  grid = E // _EBLK
  blk = lambda i: (i, 0)
  full = lambda i: (0, 0)
  out_p = pl.pallas_call(
      _tc_mlp_body,
      grid=(grid,),
      in_specs=[
          pl.BlockSpec((_EBLK // 8, 128), blk),
          pl.BlockSpec((_EBLK // 8, 128), blk),
          pl.BlockSpec((_EBLK // 8, 128), blk),
          pl.BlockSpec((128, 256), full),
          pl.BlockSpec((128, 256), full),
          pl.BlockSpec((128, 256), full),
          pl.BlockSpec((1, 256), full),
          pl.BlockSpec((256, 128), full),
          pl.BlockSpec((1, 128), full),
          pl.BlockSpec((128, 16), full),
          pl.BlockSpec((1, 16), full),
      ],
      out_specs=pl.BlockSpec((_EBLK, 2), blk),
      out_shape=jax.ShapeDtypeStruct((E, 2), jnp.float32),
  )(hs_p, hd_p, ea_p, bda, bdb, bdc, b1p, bd2, b2p, bd3, b3p)
  return out_p


def kernel(x, edge_index, edge_attr, W1, b1, g1, be1, rm1, rv1,
           W2, b2, g2, be2, rm2, rv2, W3, b3, g3, be3, rm3, rv3,
           ecW1, ecb1, ecW2, ecb2, ecW3, ecb3):
  src = edge_index[0]
  dst = edge_index[1]
  src_pad = jnp.concatenate([src, jnp.zeros((EPAD - E,), jnp.int32)])
  dst_pad = jnp.concatenate([dst, jnp.full((EPAD - E,), N, jnp.int32)])
  src_p = src_pad.reshape(NW, STEPS, CH)
  dst_p = dst_pad.reshape(NW, STEPS, CH)
  src_s = src_pad.reshape(NS, ESTEPS, CH)
  dst_s = dst_pad.reshape(NS, ESTEPS, CH)
  x_p = jnp.pad(x, ((0, NR - N), (0, 0)))
  ea_p, mlpw = _mlp_prep(edge_attr, ecW1, ecb1, ecW2, ecb2, ecW3, ecb3)

  degp = _sc_degree(dst_p)
  y1a, y1b, dis, disb, c3 = _tc_prep(degp, x_p, W1, b3, g3, be3, rm3, rv3)
  agg1 = _sc_gs64(src_s, dst_s, y1a, y1b)
  y2a, y2b = _tc_layer(agg1, y1a, y1b, dis, b1, g1, be1, rm1, rv1, W2, True)
  agg2 = _sc_gs32(src_s, dst_s, y2a, y2b)
  y3 = _tc_layer(agg2, y2a, y2b, dis, b2, g2, be2, rm2, rv2, W3, False)
  agg3p = _sc_gs16(src_p, dst_p, y3)
  hs, hd = _sc_edge(src_p, dst_p, agg3p, y3, disb, c3)
  return _tc_mlp(hs, hd, ea_p, mlpw)
